# baseline (speedup 1.0000x reference)
import functools
import math

import jax
import jax.numpy as jnp
from jax import lax
from jax.experimental import pallas as pl
from jax.experimental.pallas import tpu as pltpu
from jax.experimental.pallas import tpu_sc as plsc

N = 4
C = 64
S = 512
P = 196
PPAD = 256
BIG = 3.0e38

NC = 2
NS = 16
NW = NC * NS
LANES = 16
POS_PER_TILE = PPAD // 8


def _tc_distance_body(x_ref, cb_ref, d_ref):
    cb = cb_ref[...]
    cb2 = jnp.sum(cb * cb, axis=1, keepdims=True)
    ones_col = jnp.ones((S, 1), jnp.float32)
    cbe = jnp.concatenate([-2.0 * cb, cb2, ones_col], axis=1)
    pad = jnp.full((PPAD - P, S), BIG, jnp.float32)
    for n in range(N):
        xn = x_ref[n]
        xn2 = jnp.sum(xn * xn, axis=0, keepdims=True)
        ones_row = jnp.ones((1, P), jnp.float32)
        xe = jnp.concatenate([xn, ones_row, xn2], axis=0)
        d = jax.lax.dot_general(xe, cbe, (((0,), (1,)), ((), ())),
                                precision=jax.lax.Precision.HIGHEST,
                                preferred_element_type=jnp.float32)
        d_ref[n] = jnp.concatenate([d, pad], axis=0)


def _sc_body(d_hbm, cb_hbm, idx_hbm, xq_hbm, part_hbm,
             dv, idxv, rowsv, pv, sem):
    wid = lax.axis_index("s") * NC + lax.axis_index("c")
    n = wid // 8
    t = wid % 8
    base = t * POS_PER_TILE
    vcnt = P - base

    pltpu.sync_copy(d_hbm.at[n, pl.ds(base, POS_PER_TILE), :], dv)

    lane = lax.iota(jnp.int32, LANES)
    expw = jnp.exp(-lane.astype(jnp.float32))
    zf = jnp.zeros((LANES,), jnp.float32)
    lane0 = lane == 0

    def body(i, carry):
        acc_cb, acc_cm, ig0, ig1 = carry
        runs = []
        for j in range(S // LANES):
            kj = dv[i, pl.ds(LANES * j, LANES)]
            runs.append(plsc.sort_key_val(kj, lane + LANES * j))
        while len(runs) > 1:
            nxt = []
            for a in range(0, len(runs), 2):
                ka, va = runs[a]
                kb, vb = runs[a + 1]
                rk = lax.rev(kb, (0,))
                rv = lax.rev(vb, (0,))
                take_a = ka <= rk
                lo_k = jnp.minimum(ka, rk)
                lo_v = jnp.where(take_a, va, rv)
                nxt.append(plsc.sort_key_val(lo_k, lo_v))
            runs = nxt
        bk, bi = runs[0]

        b0 = jnp.full((LANES,), jnp.min(bk), jnp.float32)
        ti = jnp.min(jnp.where(bk == b0, bi, jnp.int32(1 << 20)))
        ti = jnp.minimum(ti, jnp.int32(S - 1))

        valid = i < vcnt
        wcb = jnp.where(valid, expw, zf)
        acc_cb = acc_cb + bk * wcb
        cm = jnp.where(lane0, bk, zf)
        acc_cm = acc_cm + jnp.where(valid, cm, zf)

        tiv = jnp.full((LANES,), ti, jnp.int32)
        m0 = lane == jnp.full((LANES,), i & (LANES - 1), jnp.int32)
        new0 = jnp.where(i < LANES, tiv, ig0)
        new1 = jnp.where(i < LANES, ig1, tiv)
        ig0 = jnp.where(m0, new0, ig0)
        ig1 = jnp.where(m0, new1, ig1)
        return acc_cb, acc_cm, ig0, ig1

    zi = jnp.zeros((LANES,), jnp.int32)
    acc_cb, acc_cm, ig0, ig1 = lax.fori_loop(
        0, POS_PER_TILE, body, (zf, zf, zi, zi))

    idxv[pl.ds(0, LANES)] = ig0
    idxv[pl.ds(LANES, LANES)] = ig1
    pv[0, :] = acc_cb
    pv[1, :] = acc_cm

    pltpu.sync_copy(idxv, idx_hbm.at[n, pl.ds(base, POS_PER_TILE)])
    pltpu.async_copy(cb_hbm.at[idxv], rowsv, sem).wait()
    pltpu.sync_copy(rowsv, xq_hbm.at[n, pl.ds(base, POS_PER_TILE), :])
    pltpu.sync_copy(pv, part_hbm.at[pl.ds(2 * wid, 2), :])


@functools.partial(
    pl.kernel,
    out_type=(
        jax.ShapeDtypeStruct((N, PPAD), jnp.int32),
        jax.ShapeDtypeStruct((N, PPAD, C), jnp.float32),
        jax.ShapeDtypeStruct((2 * NW, LANES), jnp.float32),
    ),
    mesh=plsc.VectorSubcoreMesh(
        core_axis_name="c", subcore_axis_name="s",
        num_cores=NC, num_subcores=NS),
    compiler_params=pltpu.CompilerParams(
        needs_layout_passes=False, use_tc_tiling_on_sc=False),
    scratch_types=[
        pltpu.VMEM((POS_PER_TILE, S), jnp.float32),
        pltpu.VMEM((POS_PER_TILE,), jnp.int32),
        pltpu.VMEM((POS_PER_TILE, C), jnp.float32),
        pltpu.VMEM((2, LANES), jnp.float32),
        pltpu.SemaphoreType.DMA,
    ],
)
def _sc_stage(d_hbm, cb_hbm, idx_hbm, xq_hbm, part_hbm, dv, idxv, rowsv, pv,
              sem):
    _sc_body(d_hbm, cb_hbm, idx_hbm, xq_hbm, part_hbm, dv, idxv, rowsv, pv,
             sem)


def kernel(x, codebook):
    x3 = x.reshape(N, C, P)
    d = pl.pallas_call(
        _tc_distance_body,
        out_shape=jax.ShapeDtypeStruct((N, PPAD, S), jnp.float32),
        out_specs=pl.BlockSpec(memory_space=pltpu.VMEM),
        in_specs=(
            pl.BlockSpec(memory_space=pltpu.VMEM),
            pl.BlockSpec(memory_space=pltpu.VMEM),
        ),
    )(x3, codebook)
    idx, xq, parts = _sc_stage(d, codebook)
    output = xq[:, :P, :].transpose(0, 2, 1).reshape(x.shape)
    lcb = jnp.sum(parts[0::2]) / jnp.float32(N * S * P)
    lcm = jnp.sum(parts[1::2]) / jnp.float32(N * C * P)
    return (output, lcb, lcm, idx[:, :P].reshape(N, 14, 14))


# speedup vs baseline: 1.2254x; 1.2254x over previous
import functools
import math

import jax
import jax.numpy as jnp
from jax import lax
from jax.experimental import pallas as pl
from jax.experimental.pallas import tpu as pltpu
from jax.experimental.pallas import tpu_sc as plsc

N = 4
C = 64
S = 512
P = 196
PPAD = 256
K = 16
BIG = 3.0e38
_EXPW = [math.exp(-k) for k in range(K)]

NC = 2
NS = 16
POS_PER_TILE = PPAD // 8


def _tc_distance_body(x_ref, cb_ref, d_ref, idx_ref, m0_ref, lcm_ref):
    cb = cb_ref[...]
    cb2 = jnp.sum(cb * cb, axis=1, keepdims=True)
    iota_s = jax.lax.broadcasted_iota(jnp.int32, (S, P), 0)
    zpad = jnp.zeros((1, PPAD - P), jnp.int32)
    lcm = jnp.float32(0.0)
    for n in range(N):
        xn = x_ref[n]
        xn2 = jnp.sum(xn * xn, axis=0, keepdims=True)
        dot = jax.lax.dot_general(cb, xn, (((1,), (0,)), ((), ())),
                                  precision=jax.lax.Precision.HIGHEST,
                                  preferred_element_type=jnp.float32)
        d = cb2 - 2.0 * dot + xn2
        d_ref[n] = d
        m = jnp.min(d, axis=0, keepdims=True)
        fidx = jnp.min(jnp.where(d == m, iota_s, S), axis=0, keepdims=True)
        idx_ref[pl.ds(n, 1), :] = jnp.concatenate([fidx, zpad], axis=1)
        m0_ref[pl.ds(n, 1), :] = m
        lcm = lcm + jnp.sum(m)
    lcm_ref[0, 0] = lcm / jnp.float32(N * C * P)


def _tc_loss_body(d_ref, m0_ref, lcb_ref):
    lcb = jnp.float32(0.0)
    for n in range(N):
        dw = d_ref[n]
        mk = m0_ref[pl.ds(n, 1), :]
        for k in range(K):
            lcb = lcb + jnp.float32(_EXPW[k]) * jnp.sum(mk)
            if k < K - 1:
                dw = jnp.where(dw == mk, BIG, dw)
                mk = jnp.min(dw, axis=0, keepdims=True)
    lcb_ref[0, 0] = lcb / jnp.float32(N * S * P)


def _sc_gather_body(idx_hbm, cb_hbm, xq_hbm, idxv, rowsv, sem):
    wid = lax.axis_index("s") * NC + lax.axis_index("c")
    n = wid // 8
    t = wid % 8
    base = t * POS_PER_TILE
    pltpu.sync_copy(idx_hbm.at[n, pl.ds(base, POS_PER_TILE)], idxv)
    pltpu.async_copy(cb_hbm.at[idxv], rowsv, sem).wait()
    pltpu.sync_copy(rowsv, xq_hbm.at[n, pl.ds(base, POS_PER_TILE), :])


@functools.partial(
    pl.kernel,
    out_type=jax.ShapeDtypeStruct((N, PPAD, C), jnp.float32),
    mesh=plsc.VectorSubcoreMesh(
        core_axis_name="c", subcore_axis_name="s",
        num_cores=NC, num_subcores=NS),
    compiler_params=pltpu.CompilerParams(
        needs_layout_passes=False, use_tc_tiling_on_sc=False),
    scratch_types=[
        pltpu.VMEM((POS_PER_TILE,), jnp.int32),
        pltpu.VMEM((POS_PER_TILE, C), jnp.float32),
        pltpu.SemaphoreType.DMA,
    ],
)
def _sc_gather(idx_hbm, cb_hbm, xq_hbm, idxv, rowsv, sem):
    _sc_gather_body(idx_hbm, cb_hbm, xq_hbm, idxv, rowsv, sem)


def kernel(x, codebook):
    x3 = x.reshape(N, C, P)
    d, idx, m0, lcm = pl.pallas_call(
        _tc_distance_body,
        out_shape=(
            jax.ShapeDtypeStruct((N, S, P), jnp.float32),
            jax.ShapeDtypeStruct((N, PPAD), jnp.int32),
            jax.ShapeDtypeStruct((N, P), jnp.float32),
            jax.ShapeDtypeStruct((1, 1), jnp.float32),
        ),
        out_specs=(
            pl.BlockSpec(memory_space=pltpu.VMEM),
            pl.BlockSpec(memory_space=pltpu.VMEM),
            pl.BlockSpec(memory_space=pltpu.VMEM),
            pl.BlockSpec(memory_space=pltpu.SMEM),
        ),
        in_specs=(
            pl.BlockSpec(memory_space=pltpu.VMEM),
            pl.BlockSpec(memory_space=pltpu.VMEM),
        ),
    )(x3, codebook)

    xq = _sc_gather(idx, codebook)

    lcb = pl.pallas_call(
        _tc_loss_body,
        out_shape=jax.ShapeDtypeStruct((1, 1), jnp.float32),
        out_specs=pl.BlockSpec(memory_space=pltpu.SMEM),
        in_specs=(
            pl.BlockSpec(memory_space=pltpu.VMEM),
            pl.BlockSpec(memory_space=pltpu.VMEM),
        ),
    )(d, m0)

    output = xq[:, :P, :].transpose(0, 2, 1).reshape(x.shape)
    return (output, lcb[0, 0], lcm[0, 0], idx[:, :P].reshape(N, 14, 14))
